# Initial kernel scaffold; baseline (speedup 1.0000x reference)
import jax
import jax.numpy as jnp
from jax.experimental import pallas as pl


def kernel(edge_index, responses, student_emb, item_emb, W_rel, Wg, bg, Wns, bns, Wncs, bncs, Was, bas, Wni, bni, Wnci, bnci, Wai, bai):
    raise NotImplementedError("write your pallas kernel here")


# speedup vs baseline: 2.5638x; 2.5638x over previous
import jax
import jax.numpy as jnp
from jax import lax
from jax.experimental import pallas as pl
from jax.experimental.pallas import tpu as pltpu
from jax.experimental.pallas import tpu_sc as plsc

N_NODES = 50000
EMB = 64
LANES = 128
E = 800000
C = 128
NCHUNK = E // C
CS = 64
NCHUNK_S = E // CS
NW = 32
HALF = N_NODES // 2
STRIPE = 1568
TAB = 16 * STRIPE
DUMMY = HALF
STRIPE2 = 784
TAB2 = 16 * STRIPE2
BE = 3200
DENW = 4
BN = 1000


def _gather_body(sidx_hbm, eidx_hbm, resp_hbm, s_tab, i_tab, es_out, eq_out,
                 sidx_v, eidx_v, r_v, es_buf, eq_buf, sem1, sem2):
    c = lax.axis_index("c")
    s = lax.axis_index("s")
    wid = s * 2 + c
    lane64 = jnp.full((16,), EMB, jnp.int32)
    rows16 = jnp.arange(16, dtype=jnp.int32)

    def body(j, carry):
        chunk = wid + NW * j

        @pl.when(chunk < NCHUNK)
        def _():
            base = chunk * C
            pltpu.sync_copy(sidx_hbm.at[pl.ds(base, C)], sidx_v)
            pltpu.sync_copy(eidx_hbm.at[pl.ds(base, C)], eidx_v)
            pltpu.sync_copy(resp_hbm.at[pl.ds(base, C)], r_v)
            cp1 = pltpu.async_copy(s_tab.at[sidx_v], es_buf, sem1)
            cp2 = pltpu.async_copy(i_tab.at[eidx_v], eq_buf, sem2)
            cp1.wait()
            cp2.wait()
            for k in range(C // 16):
                sl = pl.ds(16 * k, 16)
                rk = rows16 + 16 * k
                plsc.store_scatter(es_buf, [rk, lane64], r_v[sl])
                plsc.store_scatter(es_buf, [rk, lane64 + 1],
                                   (sidx_v[sl] & 1).astype(jnp.float32))
                plsc.store_scatter(eq_buf, [rk, lane64],
                                   (eidx_v[sl] & 1).astype(jnp.float32))
            pltpu.sync_copy(es_buf, es_out.at[pl.ds(base, C)])
            pltpu.sync_copy(eq_buf, eq_out.at[pl.ds(base, C)])

        return carry

    lax.fori_loop(0, (NCHUNK + NW - 1) // NW, body, 0)


def _num_scatter_body(idx_hbm, msg_hbm, num_out, idx_v, msg_v, zbuf, num_tab):
    c = lax.axis_index("c")
    s = lax.axis_index("s")
    base_node = c * HALF

    for j in range(16):
        for k in range(LANES // 16):
            zbuf[j, pl.ds(16 * k, 16)] = jnp.zeros((16,), jnp.float32)

    def zero_body(i, carry):
        pltpu.sync_copy(zbuf, num_tab.at[pl.ds(s * STRIPE2 + i * 16, 16)])
        return carry

    lax.fori_loop(0, STRIPE2 // 16, zero_body, 0)
    plsc.subcore_barrier()

    def body(j, carry):
        chunk = s + 16 * j

        @pl.when(chunk < NCHUNK_S)
        def _():
            e0 = chunk * CS
            pltpu.sync_copy(idx_hbm.at[pl.ds(e0, CS)], idx_v)
            pltpu.sync_copy(msg_hbm.at[pl.ds(e0, CS)], msg_v)
            for k in range(CS // 16):
                v = idx_v[pl.ds(16 * k, 16)] - base_node
                oob = (v < 0) | (v >= HALF)
                idx_v[pl.ds(16 * k, 16)] = jnp.where(oob, DUMMY, v) >> 1
            pltpu.sync_copy(msg_v, num_tab.at[idx_v], add=True)

        return carry

    lax.fori_loop(0, (NCHUNK_S + 15) // 16, body, 0)
    plsc.subcore_barrier()

    def out_body(i, carry):
        r0 = s * STRIPE2 + i * 16
        pltpu.sync_copy(num_tab.at[pl.ds(r0, 16)], msg_v.at[pl.ds(0, 16)])
        pltpu.sync_copy(msg_v.at[pl.ds(0, 16)], num_out.at[c, pl.ds(r0, 16)])
        return carry

    lax.fori_loop(0, STRIPE2 // 16, out_body, 0)


def _edge_body(es_ref, eq_ref, w8_ref, w2_ref,
               outs_ref, outi_ref, outds_ref, outdi_ref):
    es = es_ref[...]
    eq = eq_ref[...]
    g2 = w8_ref[0:1, :]
    bgr = w8_ref[1:2, :]
    u_es = w8_ref[2:3, :]
    u_cs = w8_ref[3:4, :]
    u_eq = w8_ref[4:5, :]
    u_css = w8_ref[5:6, :]
    c_s = w8_ref[6:7, 0:1]
    c_i = w8_ref[6:7, 1:2]
    e64 = w8_ref[7:8, :]
    w2 = w2_ref[...]

    r = es[:, EMB:EMB + 1]
    par_s = es[:, EMB + 1:EMB + 2]
    par_i = eq[:, EMB:EMB + 1]
    zhalf = jnp.zeros((es.shape[0], EMB), jnp.float32)
    base = r * g2 + bgr
    ls_q = jnp.dot(eq, w2, preferred_element_type=jnp.float32) + base
    cs_q = eq * ls_q
    a_s = (jnp.sum(es * u_es, axis=1, keepdims=True)
           + jnp.sum(cs_q * u_cs, axis=1, keepdims=True) + c_s)
    a_s = jnp.where(a_s >= 0, a_s, 0.01 * a_s)
    w_s = jnp.exp(a_s)

    ls_s = jnp.dot(es, w2, preferred_element_type=jnp.float32) + base
    cs_s = es * ls_s
    a_i = (jnp.sum(eq * u_eq, axis=1, keepdims=True)
           + jnp.sum(cs_s * u_css, axis=1, keepdims=True) + c_i)
    a_i = jnp.where(a_i >= 0, a_i, 0.01 * a_i)
    w_i = jnp.exp(a_i)

    m_s = w_s * cs_q
    m_s_hi = jnp.concatenate([zhalf, m_s[:, :EMB]], axis=1)
    outs_ref[...] = m_s * (1.0 - par_s) + m_s_hi * par_s
    m_i = w_i * cs_s
    m_i_hi = jnp.concatenate([zhalf, m_i[:, :EMB]], axis=1)
    outi_ref[...] = m_i * (1.0 - par_i) + m_i_hi * par_i
    lane = lax.broadcasted_iota(jnp.int32, (es.shape[0], LANES), 1)
    sel_s = jnp.where(lane == (par_s * EMB).astype(jnp.int32), 1.0, 0.0)
    sel_i = jnp.where(lane == (par_i * EMB).astype(jnp.int32), 1.0, 0.0)
    outds_ref[...] = w_s * sel_s
    outdi_ref[...] = w_i * sel_i


def _final_body(num_ref, den_ref, embe_ref, embo_ref, oute_ref, outo_ref):
    nb = num_ref[0]
    dd = den_ref[0]
    de = dd[:, 0:1]
    de = jnp.where(de == 0.0, 1.0, de)
    do = dd[:, EMB:EMB + 1]
    do = jnp.where(do == 0.0, 1.0, do)
    oute_ref[...] = (embe_ref[0] + nb[:, :EMB] / de)[None]
    outo_ref[...] = (embo_ref[0] + nb[:, EMB:] / do)[None]


def kernel(edge_index, responses, student_emb, item_emb, W_rel, Wg, bg,
           Wns, bns, Wncs, bncs, Was, bas, Wni, bni, Wnci, bnci, Wai, bai):
    s_idx = edge_index[0].astype(jnp.int32)
    e_idx = edge_index[1].astype(jnp.int32)
    resp = responses.astype(jnp.float32)

    padn = jnp.zeros((N_NODES, LANES - EMB), jnp.float32)
    stud128 = jnp.concatenate([student_emb, padn], axis=1)
    item128 = jnp.concatenate([item_emb, padn], axis=1)

    def pad128(v):
        return jnp.concatenate([v, jnp.zeros((LANES - EMB,), jnp.float32)])

    g2 = pad128(Wg[:, EMB:] @ W_rel[:, 0])
    bgr = pad128(bg)
    u_es = pad128(Wns.T @ Was[0, :EMB])
    u_cs = pad128(Wncs.T @ Was[0, EMB:])
    c_s = bns @ Was[0, :EMB] + bncs @ Was[0, EMB:] + bas[0]
    u_eq = pad128(Wni.T @ Wai[0, :EMB])
    u_css = pad128(Wnci.T @ Wai[0, EMB:])
    c_i = bni @ Wai[0, :EMB] + bnci @ Wai[0, EMB:] + bai[0]
    row6 = jnp.zeros((LANES,), jnp.float32).at[0].set(c_s).at[1].set(c_i)
    e64 = jnp.zeros((LANES,), jnp.float32).at[EMB].set(1.0)
    w8 = jnp.stack([g2, bgr, u_es, u_cs, u_eq, u_css, row6, e64])
    w2 = jnp.zeros((LANES, LANES), jnp.float32).at[:EMB, :EMB].set(Wg[:, :EMB].T)

    mesh = plsc.VectorSubcoreMesh(core_axis_name="c", subcore_axis_name="s")

    gather = pl.kernel(
        _gather_body, mesh=mesh,
        compiler_params=pltpu.CompilerParams(needs_layout_passes=False),
        out_type=[jax.ShapeDtypeStruct((E, LANES), jnp.float32),
                  jax.ShapeDtypeStruct((E, LANES), jnp.float32)],
        scratch_types=[pltpu.VMEM((C,), jnp.int32),
                       pltpu.VMEM((C,), jnp.int32),
                       pltpu.VMEM((C,), jnp.float32),
                       pltpu.VMEM((C, LANES), jnp.float32),
                       pltpu.VMEM((C, LANES), jnp.float32),
                       pltpu.SemaphoreType.DMA,
                       pltpu.SemaphoreType.DMA])
    es_on, eq_on = gather(s_idx, e_idx, resp, stud128, item128)

    nblk = E // BE
    edge_fn = pl.pallas_call(
        _edge_body,
        grid=(nblk,),
        in_specs=[pl.BlockSpec((BE, LANES), lambda i: (i, 0)),
                  pl.BlockSpec((BE, LANES), lambda i: (i, 0)),
                  pl.BlockSpec((8, LANES), lambda i: (0, 0)),
                  pl.BlockSpec((LANES, LANES), lambda i: (0, 0))],
        out_specs=[pl.BlockSpec((BE, LANES), lambda i: (i, 0)),
                   pl.BlockSpec((BE, LANES), lambda i: (i, 0)),
                   pl.BlockSpec((BE, LANES), lambda i: (i, 0)),
                   pl.BlockSpec((BE, LANES), lambda i: (i, 0))],
        out_shape=[jax.ShapeDtypeStruct((E, LANES), jnp.float32),
                   jax.ShapeDtypeStruct((E, LANES), jnp.float32),
                   jax.ShapeDtypeStruct((E, LANES), jnp.float32),
                   jax.ShapeDtypeStruct((E, LANES), jnp.float32)])
    msg_s, msg_i, dw_s, dw_i = edge_fn(es_on, eq_on, w8, w2)

    nscat = pl.kernel(
        _num_scatter_body, mesh=mesh,
        out_type=jax.ShapeDtypeStruct((2, TAB2, LANES), jnp.float32),
        scratch_types=[pltpu.VMEM((CS,), jnp.int32),
                       pltpu.VMEM((CS, LANES), jnp.float32),
                       pltpu.VMEM((16, LANES), jnp.float32),
                       pltpu.VMEM_SHARED((TAB2, LANES), jnp.float32)])
    num_s = nscat(s_idx, msg_s)
    num_i = nscat(e_idx, msg_i)
    den_s = nscat(s_idx, dw_s)
    den_i = nscat(e_idx, dw_i)

    QH = HALF // 2
    BN2 = 784
    final_fn = pl.pallas_call(
        _final_body,
        grid=(2, TAB2 // BN2),
        in_specs=[pl.BlockSpec((1, BN2, LANES), lambda c, j: (c, j, 0)),
                  pl.BlockSpec((1, BN2, LANES), lambda c, j: (c, j, 0)),
                  pl.BlockSpec((1, BN2, EMB), lambda c, j: (c, j, 0)),
                  pl.BlockSpec((1, BN2, EMB), lambda c, j: (c, j, 0))],
        out_specs=[pl.BlockSpec((1, BN2, EMB), lambda c, j: (c, j, 0)),
                   pl.BlockSpec((1, BN2, EMB), lambda c, j: (c, j, 0))],
        out_shape=[jax.ShapeDtypeStruct((2, TAB2, EMB), jnp.float32),
                   jax.ShapeDtypeStruct((2, TAB2, EMB), jnp.float32)])

    def pad3(x):
        return jnp.zeros((2, TAB2, EMB), jnp.float32).at[:, :QH, :].set(
            x.reshape(2, QH, EMB))

    def finish(num, den, emb):
        oute3, outo3 = final_fn(num, den, pad3(emb[0::2]), pad3(emb[1::2]))
        oute = oute3[:, :QH].reshape(N_NODES // 2, EMB)
        outo = outo3[:, :QH].reshape(N_NODES // 2, EMB)
        return jnp.stack([oute, outo], axis=1).reshape(N_NODES, EMB)

    es_updated = finish(num_s, den_s, student_emb)
    eq_updated = finish(num_i, den_i, item_emb)
    return (es_updated, eq_updated)
